# baseline (speedup 1.0000x reference)
import dataclasses
import functools

import jax
import jax.numpy as jnp
from jax import lax
from jax.experimental import pallas as pl
from jax.experimental.pallas import tpu as pltpu
from jax.experimental.pallas import tpu_sc as plsc

_BS, _NP, _K, _DIN, _D, _E = 4, 2500, 8, 16, 128, 320000
_N = _BS * _NP
_NC, _NS = 2, 16
_NW = _NC * _NS
_TBL = 320
_HP = _NW * _TBL
_ACC = 10008
_RPT = 624
_TRASH = _N
_CAP = 24576
_CHUNK = 8000
_NCHUNK = _E // _CHUNK
_GRP = 128
_NGMAX = _CAP // _GRP
_BLK = 16
_NBG = _GRP // _BLK


def _sc_params():
    cp = pltpu.CompilerParams()
    if "needs_layout_passes" in pltpu.CompilerParams.__dataclass_fields__:
        cp = dataclasses.replace(cp, needs_layout_passes=False)
    return cp


def _tc_pre(pcd, c2, W_flat, b_flat, W_fc, b_fc):
    def body(pcd_ref, c2_ref, wf_ref, bf_ref, wfc_ref, bfc_ref, o_ref):
        wbig = jnp.tile(wf_ref[...], (_K, 1)) * (1.0 / _K)
        flat = jnp.dot(c2_ref[...], wbig,
                       preferred_element_type=jnp.float32) + bf_ref[...]
        pcd_v = pcd_ref[...]
        cic = (pcd_v - jnp.mean(pcd_v, axis=1, keepdims=True) - 1.0)
        cic = cic.reshape(_N, 2)
        x = jnp.dot(flat, wfc_ref[0:_D, :], preferred_element_type=jnp.float32)
        x = x + cic[:, 0:1] * wfc_ref[_D:_D + 1, :]
        x = x + cic[:, 1:2] * wfc_ref[_D + 1:_D + 2, :]
        o_ref[...] = x + bfc_ref[...]

    return pl.pallas_call(
        body,
        out_shape=jax.ShapeDtypeStruct((_N, _D), jnp.float32),
    )(pcd, c2, W_flat, b_flat, W_fc, b_fc)


def _sc_bin(src1d, dst1d):
    mesh = plsc.VectorSubcoreMesh(core_axis_name="c", subcore_axis_name="s")

    @functools.partial(
        pl.kernel,
        out_type=(
            jax.ShapeDtypeStruct((_NC, _NS, _CAP), jnp.int32),
            jax.ShapeDtypeStruct((_NC, _NS, _CAP), jnp.int32),
            jax.ShapeDtypeStruct((_NC, _NS, 16), jnp.int32),
        ),
        mesh=mesh,
        compiler_params=_sc_params(),
        scratch_types=[
            pltpu.VMEM((_CHUNK,), jnp.int32),
            pltpu.VMEM((_CHUNK,), jnp.int32),
            pltpu.VMEM((_CHUNK,), jnp.int32),
            pltpu.VMEM((_CHUNK,), jnp.int32),
            pltpu.VMEM((_CAP,), jnp.int32),
            pltpu.VMEM((_CAP,), jnp.int32),
            pltpu.VMEM((16,), jnp.int32),
            pltpu.SMEM((1,), jnp.int32),
            pltpu.SemaphoreType.DMA,
            pltpu.SemaphoreType.DMA,
        ],
    )
    def k(src_hbm, dst_hbm, elsrc_hbm, eldst_hbm, cnt_hbm,
          sb0, sb1, db0, db1, csrc, cdst, ncb, off_ref, sem0, sem1):
        cid = lax.axis_index("c")
        sid = lax.axis_index("s")
        wid = cid * _NS + sid
        lo = wid * _TBL
        lo_v = jnp.full((16,), lo, jnp.int32)
        hi_v = lo_v + _TBL

        @pl.loop(0, _CAP, step=16)
        def _(i):
            csrc.at[pl.ds(i, 16)][...] = lo_v
            cdst.at[pl.ds(i, 16)][...] = jnp.full((16,), _TRASH, jnp.int32)

        off_ref[0] = 0
        sbufs, dbufs, sems = (sb0, sb1), (db0, db1), (sem0, sem1)
        for q in range(2):
            pltpu.async_copy(src_hbm.at[pl.ds(q * _CHUNK, _CHUNK)], sbufs[q],
                             sems[q])
            pltpu.async_copy(dst_hbm.at[pl.ds(q * _CHUNK, _CHUNK)], dbufs[q],
                             sems[q])

        @pl.loop(0, _NCHUNK, step=2)
        def _(g):
            for u in range(2):
                c = g + u
                pltpu.make_async_copy(src_hbm.at[pl.ds(c * _CHUNK, _CHUNK)],
                                      sbufs[u], sems[u]).wait()
                pltpu.make_async_copy(dst_hbm.at[pl.ds(c * _CHUNK, _CHUNK)],
                                      dbufs[u], sems[u]).wait()

                @pl.loop(0, _CHUNK, step=16)
                def _(i):
                    s = sbufs[u].at[pl.ds(i, 16)][...]
                    d = dbufs[u].at[pl.ds(i, 16)][...]
                    m = (s >= lo_v) & (s < hi_v)
                    c16 = plsc.all_reduce_population_count(m)
                    o = off_ref[0]
                    plsc.store_compressed(csrc.at[pl.ds(o, 16)], s, mask=m)
                    plsc.store_compressed(cdst.at[pl.ds(o, 16)], d, mask=m)
                    off_ref[0] = jnp.minimum(o + c16[0], _CAP - 16)

                @pl.when(c + 2 < _NCHUNK)
                def _():
                    pltpu.async_copy(
                        src_hbm.at[pl.ds((c + 2) * _CHUNK, _CHUNK)],
                        sbufs[u], sems[u])
                    pltpu.async_copy(
                        dst_hbm.at[pl.ds((c + 2) * _CHUNK, _CHUNK)],
                        dbufs[u], sems[u])

        cnt = off_ref[0]
        ncb[...] = jnp.full((16,), cnt, jnp.int32)
        pltpu.sync_copy(ncb, cnt_hbm.at[cid, sid])
        pltpu.sync_copy(csrc, elsrc_hbm.at[cid, sid])
        pltpu.sync_copy(cdst, eldst_hbm.at[cid, sid])

    return k(src1d, dst1d)


def _sc_layer(hp, elsrc, eldst, counts, zrows):
    mesh = plsc.VectorSubcoreMesh(core_axis_name="c", subcore_axis_name="s")

    @functools.partial(
        pl.kernel,
        out_type=jax.ShapeDtypeStruct((_NC, _ACC, _D), jnp.float32),
        mesh=mesh,
        compiler_params=_sc_params(),
        scratch_types=[
            pltpu.VMEM((_TBL, _D), jnp.float32),
            pltpu.VMEM((_BLK, _D), jnp.float32),
            pltpu.VMEM((_BLK, _D), jnp.float32),
            pltpu.VMEM((_GRP,), jnp.int32),
            pltpu.VMEM((_GRP,), jnp.int32),
            pltpu.VMEM((_GRP,), jnp.int32),
            pltpu.VMEM((_GRP,), jnp.int32),
            pltpu.VMEM((_NBG, _BLK), jnp.int32),
            pltpu.VMEM((_NBG, _BLK), jnp.int32),
            pltpu.VMEM((16,), jnp.int32),
            pltpu.VMEM_SHARED((_ACC, _D), jnp.float32),
            pltpu.SemaphoreType.DMA,
            pltpu.SemaphoreType.DMA,
            pltpu.SemaphoreType.DMA,
            pltpu.SemaphoreType.DMA,
            pltpu.SemaphoreType.DMA,
        ],
    )
    def k(hp_hbm, elsrc_hbm, eldst_hbm, cnt_hbm, z_hbm, out_hbm,
          tbuf, rb0, rb1, ss0, ss1, df0, df1, ds0, ds1, ncb, acc,
          tsem, gsem0, gsem1, wsem0, wsem1):
        cid = lax.axis_index("c")
        sid = lax.axis_index("s")
        wid = cid * _NS + sid
        lo = wid * _TBL
        lo_v = jnp.full((16,), lo, jnp.int32)
        iota16 = lax.iota(jnp.int32, 16)
        r0 = sid * _RPT
        sstg, dflt, gsems = (ss0, ss1), (df0, df1), (gsem0, gsem1)
        dstg = (ds0, ds1)
        rbufs, wsems = (rb0, rb1), (wsem0, wsem1)

        pltpu.async_copy(hp_hbm.at[pl.ds(lo, _TBL)], tbuf, tsem)
        pltpu.sync_copy(z_hbm.at[pl.ds(0, _RPT)], acc.at[pl.ds(r0, _RPT)])

        @pl.when(sid == _NS - 1)
        def _():
            pltpu.sync_copy(z_hbm.at[pl.ds(0, 24)],
                            acc.at[pl.ds(_NS * _RPT, 24)])

        pltpu.sync_copy(cnt_hbm.at[cid, sid], ncb)
        cnt = ncb[pl.ds(0, 16)][0]
        ngrp = jnp.maximum((cnt + (_GRP - 1)) >> 7, 2)

        for q in range(2):
            pltpu.async_copy(elsrc_hbm.at[cid, sid, pl.ds(q * _GRP, _GRP)],
                             sstg[q], gsems[q])
            pltpu.async_copy(eldst_hbm.at[cid, sid, pl.ds(q * _GRP, _GRP)],
                             dflt[q], gsems[q])
        pltpu.make_async_copy(hp_hbm.at[pl.ds(lo, _TBL)], tbuf, tsem).wait()
        plsc.subcore_barrier()

        @pl.loop(0, _NGMAX, step=2)
        def _(g):
            for u in range(2):
                gg = g + u

                @pl.when(gg < ngrp)
                def _():
                    pltpu.make_async_copy(
                        elsrc_hbm.at[cid, sid, pl.ds(gg * _GRP, _GRP)],
                        sstg[u], gsems[u]).wait()
                    pltpu.make_async_copy(
                        eldst_hbm.at[cid, sid, pl.ds(gg * _GRP, _GRP)],
                        dflt[u], gsems[u]).wait()

                    @pl.loop(0, _NBG)
                    def _(r):
                        dstg[u].at[r, pl.ds(0, 16)][...] = (
                            dflt[u][pl.ds(r * _BLK, 16)])

                    for j in range(_NBG):
                        p = j % 2

                        if j < 2:
                            @pl.when(gg > 0)
                            def _():
                                pltpu.make_async_copy(
                                    rbufs[p], acc.at[dstg[u].at[j]],
                                    wsems[p]).wait()
                        else:
                            pltpu.make_async_copy(
                                rbufs[p], acc.at[dstg[u].at[j]],
                                wsems[p]).wait()

                        sv = sstg[u][pl.ds(j * _BLK, 16)]
                        rowv = sv - lo_v
                        for col in range(_D):
                            colv = jnp.full((16,), col, jnp.int32)
                            vals = plsc.load_gather(tbuf, [rowv, colv])
                            plsc.store_scatter(rbufs[p], [iota16, colv],
                                               vals)

                        pltpu.async_copy(rbufs[p], acc.at[dstg[u].at[j]],
                                         wsems[p], add=True)

                    @pl.when(gg + 2 < ngrp)
                    def _():
                        pltpu.async_copy(
                            elsrc_hbm.at[cid, sid,
                                         pl.ds((gg + 2) * _GRP, _GRP)],
                            sstg[u], gsems[u])
                        pltpu.async_copy(
                            eldst_hbm.at[cid, sid,
                                         pl.ds((gg + 2) * _GRP, _GRP)],
                            dflt[u], gsems[u])

        for p in range(2):
            pltpu.make_async_copy(rbufs[p], acc.at[dstg[0].at[p]],
                                  wsems[p]).wait()
        plsc.subcore_barrier()

        pltpu.sync_copy(acc.at[pl.ds(r0, _RPT)],
                        out_hbm.at[cid, pl.ds(r0, _RPT)])

        @pl.when(sid == _NS - 1)
        def _():
            pltpu.sync_copy(acc.at[pl.ds(_NS * _RPT, 24)],
                            out_hbm.at[cid, pl.ds(_NS * _RPT, 24)])

    return k(hp, elsrc, eldst, counts, zrows)


def _tc_rehome(elsrc, eldst, counts):
    def body(a_ref, b_ref, c_ref, oa_ref, ob_ref, oc_ref):
        oa_ref[...] = a_ref[...]
        ob_ref[...] = b_ref[...]
        oc_ref[...] = c_ref[...]

    return pl.pallas_call(
        body,
        out_shape=(
            jax.ShapeDtypeStruct(elsrc.shape, elsrc.dtype),
            jax.ShapeDtypeStruct(eldst.shape, eldst.dtype),
            jax.ShapeDtypeStruct(counts.shape, counts.dtype),
        ),
    )(elsrc, eldst, counts)


def _tc_update(parts, h, W, b, out_rows):
    rb = 2000

    def body(p_ref, h_ref, w_ref, b_ref, o_ref):
        agg = p_ref[0] + p_ref[1]
        z = jnp.dot(agg, w_ref[...], preferred_element_type=jnp.float32)
        o_ref[...] = jnp.maximum(z + b_ref[...], 0.0) + h_ref[...]

    return pl.pallas_call(
        body,
        grid=(_N // rb,),
        in_specs=[
            pl.BlockSpec((_NC, rb, _D), lambda i: (0, i, 0)),
            pl.BlockSpec((rb, _D), lambda i: (i, 0)),
            pl.BlockSpec((_D, _D), lambda i: (0, 0)),
            pl.BlockSpec((1, _D), lambda i: (0, 0)),
        ],
        out_specs=pl.BlockSpec((rb, _D), lambda i: (i, 0)),
        out_shape=jax.ShapeDtypeStruct((out_rows, _D), jnp.float32),
    )(parts, h, W, b)


def kernel(pcd, c_input, edge_index, W_flat, b_flat, W_fc, b_fc,
           Wg1, bg1, Wg2, bg2, Wg3, bg3):
    c2 = c_input.reshape(_N, _K * _DIN)
    x = _tc_pre(pcd, c2, W_flat, b_flat.reshape(1, _D),
                W_fc, b_fc.reshape(1, _D))
    h = jnp.pad(x, ((0, _HP - _N), (0, 0)))

    elsrc, eldst, counts = _sc_bin(edge_index[0], edge_index[1])
    elsrc, eldst, counts = _tc_rehome(elsrc, eldst, counts)
    zrows = jnp.zeros((_RPT + 24, _D), jnp.float32)

    layers = ((Wg1, bg1), (Wg2, bg2), (Wg3, bg3))
    for li, (Wg, bg) in enumerate(layers):
        parts = _sc_layer(h, elsrc, eldst, counts, zrows)
        out_rows = _N if li == len(layers) - 1 else _HP
        h = _tc_update(parts, h, Wg, bg.reshape(1, _D), out_rows)

    return h.reshape(_BS, _NP, _D)


# speedup vs baseline: 3.4237x; 3.4237x over previous
import functools

import jax
import jax.numpy as jnp
from jax import lax
from jax.experimental import pallas as pl
from jax.experimental.pallas import tpu as pltpu
from jax.experimental.pallas import tpu_sc as plsc

_BS, _NP, _K, _DIN, _D, _E = 4, 2500, 8, 16, 128, 320000
_N = _BS * _NP
_NC, _NS = 2, 16
_NW = _NC * _NS
_BLK = 128
_BPT = 80
_EPAD = _NW * _BPT * _BLK
_NBLK = _NW * _BPT
_RPT = 640
_ACC_ROWS = _NS * _RPT
_ZR = 64


def _tc_pre(pcd, c2, W_flat, b_flat, W_fc, b_fc):
    def body(pcd_ref, c2_ref, wf_ref, bf_ref, wfc_ref, bfc_ref, o_ref):
        wbig = jnp.tile(wf_ref[...], (_K, 1)) * (1.0 / _K)
        flat = jnp.dot(c2_ref[...], wbig,
                       preferred_element_type=jnp.float32) + bf_ref[...]
        pcd_v = pcd_ref[...]
        cic = (pcd_v - jnp.mean(pcd_v, axis=1, keepdims=True) - 1.0)
        cic = cic.reshape(_N, 2)
        x = jnp.dot(flat, wfc_ref[0:_D, :], preferred_element_type=jnp.float32)
        x = x + cic[:, 0:1] * wfc_ref[_D:_D + 1, :]
        x = x + cic[:, 1:2] * wfc_ref[_D + 1:_D + 2, :]
        o_ref[...] = x + bfc_ref[...]

    return pl.pallas_call(
        body,
        out_shape=jax.ShapeDtypeStruct((_N, _D), jnp.float32),
    )(pcd, c2, W_flat, b_flat, W_fc, b_fc)


def _sc_segment_sum(h, idx3d):
    mesh = plsc.VectorSubcoreMesh(core_axis_name="c", subcore_axis_name="s")

    @functools.partial(
        pl.kernel,
        out_type=jax.ShapeDtypeStruct((_NC, _ACC_ROWS, _D), jnp.float32),
        mesh=mesh,
        scratch_types=[
            pltpu.VMEM((2, _BLK), jnp.int32),
            pltpu.VMEM((2, _BLK), jnp.int32),
            pltpu.VMEM((2, _BLK), jnp.int32),
            pltpu.VMEM((2, _BLK), jnp.int32),
            pltpu.VMEM((_BLK, _D), jnp.float32),
            pltpu.VMEM((_BLK, _D), jnp.float32),
            pltpu.VMEM((_ZR, _D), jnp.float32),
            pltpu.VMEM_SHARED((_ACC_ROWS, _D), jnp.float32),
            pltpu.SemaphoreType.DMA,
            pltpu.SemaphoreType.DMA,
            pltpu.SemaphoreType.DMA,
            pltpu.SemaphoreType.DMA,
            pltpu.SemaphoreType.DMA,
            pltpu.SemaphoreType.DMA,
        ],
    )
    def k(h_hbm, idx_hbm, out_hbm,
          ib0, ib1, ib2, ib3, gb0, gb1, zbuf, acc,
          is0, is1, is2, is3, gs0, gs1):
        cid = lax.axis_index("c")
        sid = lax.axis_index("s")
        wid = cid * _NS + sid
        base = wid * _BPT
        ibufs = (ib0, ib1, ib2, ib3)
        isems = (is0, is1, is2, is3)
        gbufs = (gb0, gb1)
        gsems = (gs0, gs1)

        zv = jnp.zeros((16,), jnp.float32)

        @pl.loop(0, _ZR)
        def _(i):
            @pl.loop(0, _D, step=16)
            def _(j):
                zbuf.at[i, pl.ds(j, 16)][...] = zv

        @pl.loop(0, _RPT // _ZR)
        def _(z):
            pltpu.sync_copy(zbuf, acc.at[pl.ds(sid * _RPT + z * _ZR, _ZR)])

        plsc.subcore_barrier()

        for q in range(4):
            pltpu.make_async_copy(idx_hbm.at[base + q], ibufs[q],
                                  isems[q]).start()
        pltpu.make_async_copy(idx_hbm.at[base], ibufs[0], isems[0]).wait()
        pltpu.make_async_copy(h_hbm.at[ibufs[0].at[0]], gbufs[0],
                              gsems[0]).start()

        @pl.loop(0, _BPT, step=4)
        def _(g):
            for u in range(4):
                b = g + u
                p = u % 2
                pn = (u + 1) % 2
                qn = (u + 1) % 4

                @pl.when(b + 1 < _BPT)
                def _():
                    pltpu.make_async_copy(idx_hbm.at[base + b + 1],
                                          ibufs[qn], isems[qn]).wait()
                    pltpu.make_async_copy(h_hbm.at[ibufs[qn].at[0]],
                                          gbufs[pn], gsems[pn]).start()

                pltpu.make_async_copy(h_hbm.at[ibufs[u].at[0]], gbufs[p],
                                      gsems[p]).wait()
                pltpu.sync_copy(gbufs[p], acc.at[ibufs[u].at[1]], add=True)

                @pl.when(b + 4 < _BPT)
                def _():
                    pltpu.make_async_copy(idx_hbm.at[base + b + 4],
                                          ibufs[u], isems[u]).start()

        plsc.subcore_barrier()

        pltpu.sync_copy(acc.at[pl.ds(sid * _RPT, _RPT)],
                        out_hbm.at[cid, pl.ds(sid * _RPT, _RPT)])

    return k(h, idx3d)


def _tc_update(parts, h, W, b):
    rb = 2000

    def body(p_ref, h_ref, w_ref, b_ref, o_ref):
        agg = p_ref[0] + p_ref[1]
        z = jnp.dot(agg, w_ref[...], preferred_element_type=jnp.float32)
        o_ref[...] = jnp.maximum(z + b_ref[...], 0.0) + h_ref[...]

    return pl.pallas_call(
        body,
        grid=(_N // rb,),
        in_specs=[
            pl.BlockSpec((_NC, rb, _D), lambda i: (0, i, 0)),
            pl.BlockSpec((rb, _D), lambda i: (i, 0)),
            pl.BlockSpec((_D, _D), lambda i: (0, 0)),
            pl.BlockSpec((1, _D), lambda i: (0, 0)),
        ],
        out_specs=pl.BlockSpec((rb, _D), lambda i: (i, 0)),
        out_shape=jax.ShapeDtypeStruct((_N, _D), jnp.float32),
    )(parts, h, W, b)


def kernel(pcd, c_input, edge_index, W_flat, b_flat, W_fc, b_fc,
           Wg1, bg1, Wg2, bg2, Wg3, bg3):
    c2 = c_input.reshape(_N, _K * _DIN)
    h = _tc_pre(pcd, c2, W_flat, b_flat.reshape(1, _D),
                W_fc, b_fc.reshape(1, _D))

    pad = _EPAD - _E
    src = jnp.concatenate([edge_index[0], jnp.zeros((pad,), edge_index.dtype)])
    dst = jnp.concatenate([edge_index[1], jnp.full((pad,), _N, edge_index.dtype)])
    idx3d = jnp.stack([src.reshape(_NBLK, _BLK), dst.reshape(_NBLK, _BLK)],
                      axis=1)

    for Wg, bg in ((Wg1, bg1), (Wg2, bg2), (Wg3, bg3)):
        parts = _sc_segment_sum(h, idx3d)
        h = _tc_update(parts, h, Wg, bg.reshape(1, _D))

    return h.reshape(_BS, _NP, _D)
